# Initial kernel scaffold; baseline (speedup 1.0000x reference)
import jax
import jax.numpy as jnp
from jax.experimental import pallas as pl


def kernel(x, edge_index, kernel_id, W, gamma, beta):
    raise NotImplementedError("write your pallas kernel here")


# speedup vs baseline: 2.4871x; 2.4871x over previous
import functools

import jax
import jax.numpy as jnp
from jax import lax
from jax.experimental import pallas as pl
from jax.experimental.pallas import tpu as pltpu
from jax.experimental.pallas import tpu_sc as plsc

N = 10000
E = 320000
C = 128
K = 27
EPS = 1e-5

NUM_CORES = 2
NUM_SUBCORES = 16
NW = NUM_CORES * NUM_SUBCORES
EPW = E // NW
CHUNK = 80
NCH = EPW // CHUNK
BCH = 25
NBLK_E = NCH // BCH
ZB = 80
NZB = N // ZB

BN_ROWS = 1000
NBLK = N // BN_ROWS


def _xw_matmul(x, W):
    def body(x_ref, w_ref, o_ref):
        o_ref[...] = jnp.dot(x_ref[...], w_ref[0],
                             preferred_element_type=jnp.float32)

    return pl.pallas_call(
        body,
        grid=(NBLK, K),
        in_specs=[
            pl.BlockSpec((BN_ROWS, C), lambda b, k: (b, 0)),
            pl.BlockSpec((1, C, C), lambda b, k: (k, 0, 0)),
        ],
        out_specs=pl.BlockSpec((BN_ROWS, C), lambda b, k: (k * NBLK + b, 0)),
        out_shape=jax.ShapeDtypeStruct((K * N, C), jnp.float32),
    )(x, W)


def _sc_edge_scatter(xw, srcr, dstr, kidr, zrows):
    mesh = plsc.VectorSubcoreMesh(core_axis_name="c", subcore_axis_name="s")

    @functools.partial(
        pl.kernel,
        mesh=mesh,
        out_type=jax.ShapeDtypeStruct((NUM_CORES, N, C), jnp.float32),
        scratch_types=[
            pltpu.VMEM((BCH, CHUNK), jnp.int32),
            pltpu.VMEM((BCH, CHUNK), jnp.int32),
            pltpu.VMEM((BCH, CHUNK), jnp.int32),
            pltpu.VMEM((CHUNK, C), jnp.float32),
            pltpu.VMEM_SHARED((N, C), jnp.float32),
            pltpu.SemaphoreType.DMA,
        ],
    )
    def k(xw_hbm, src_hbm, dst_hbm, kid_hbm, z_hbm, out_hbm,
          src2, dst2, kid2, rows, acc, sem):
        cid = lax.axis_index("c")
        sid = lax.axis_index("s")
        wid = sid * NUM_CORES + cid

        for t in range(NZB // NUM_SUBCORES + 1):
            b = sid + t * NUM_SUBCORES

            @pl.when(b < NZB)
            def _():
                pltpu.sync_copy(z_hbm.at[pl.ds(b * ZB, ZB)],
                                acc.at[pl.ds(b * ZB, ZB)])

        plsc.subcore_barrier()

        def block(bi, carry0):
            pltpu.sync_copy(src_hbm.at[wid, bi], src2)
            pltpu.sync_copy(dst_hbm.at[wid, bi], dst2)
            pltpu.sync_copy(kid_hbm.at[wid, bi], kid2)

            def gidx(i, carry):
                for j in range(CHUNK // 16):
                    sl = pl.ds(j * 16, 16)
                    src2[i, sl] = kid2[i, sl] * N + src2[i, sl]
                return carry
            lax.fori_loop(0, BCH, gidx, 0)

            def step(ci, carry):
                pltpu.async_copy(xw_hbm.at[src2.at[ci]], rows, sem).wait()
                pltpu.sync_copy(rows, acc.at[dst2.at[ci]], add=True)
                return carry
            lax.fori_loop(0, BCH, step, 0)
            return carry0
        lax.fori_loop(0, NBLK_E, block, 0)

        plsc.subcore_barrier()

        for t in range(NZB // NUM_SUBCORES + 1):
            b = sid + t * NUM_SUBCORES

            @pl.when(b < NZB)
            def _():
                pltpu.sync_copy(acc.at[pl.ds(b * ZB, ZB)],
                                out_hbm.at[cid, pl.ds(b * ZB, ZB)])

    return k(xw, srcr, dstr, kidr, zrows)


def _sum_and_stats(partial):
    def body(p_ref, s_ref, st_ref):
        b = pl.program_id(0)
        s = p_ref[0] + p_ref[1]
        s_ref[...] = s

        @pl.when(b == 0)
        def _():
            st_ref[...] = jnp.zeros_like(st_ref)

        st_ref[0, :] += jnp.sum(s, axis=0)
        st_ref[1, :] += jnp.sum(s * s, axis=0)

    return pl.pallas_call(
        body,
        grid=(NBLK,),
        in_specs=[pl.BlockSpec((NUM_CORES, BN_ROWS, C), lambda b: (0, b, 0))],
        out_specs=[
            pl.BlockSpec((BN_ROWS, C), lambda b: (b, 0)),
            pl.BlockSpec((8, C), lambda b: (0, 0)),
        ],
        out_shape=[
            jax.ShapeDtypeStruct((N, C), jnp.float32),
            jax.ShapeDtypeStruct((8, C), jnp.float32),
        ],
    )(partial)


def _bn_relu(s, stats, gamma2, beta2):
    def body(s_ref, st_ref, g_ref, b_ref, o_ref):
        mean = st_ref[0:1, :] * (1.0 / N)
        var = st_ref[1:2, :] * (1.0 / N) - mean * mean
        scale = g_ref[...] * lax.rsqrt(var + EPS)
        o_ref[...] = jnp.maximum((s_ref[...] - mean) * scale + b_ref[...], 0.0)

    return pl.pallas_call(
        body,
        grid=(NBLK,),
        in_specs=[
            pl.BlockSpec((BN_ROWS, C), lambda b: (b, 0)),
            pl.BlockSpec((8, C), lambda b: (0, 0)),
            pl.BlockSpec((1, C), lambda b: (0, 0)),
            pl.BlockSpec((1, C), lambda b: (0, 0)),
        ],
        out_specs=pl.BlockSpec((BN_ROWS, C), lambda b: (b, 0)),
        out_shape=jax.ShapeDtypeStruct((N, C), jnp.float32),
    )(s, stats, gamma2, beta2)


def kernel(x, edge_index, kernel_id, W, gamma, beta):
    srcr = edge_index[0].astype(jnp.int32).reshape(NW, NBLK_E, BCH, CHUNK)
    dstr = edge_index[1].astype(jnp.int32).reshape(NW, NBLK_E, BCH, CHUNK)
    kidr = kernel_id.astype(jnp.int32).reshape(NW, NBLK_E, BCH, CHUNK)
    zrows = jnp.zeros((N, C), jnp.float32)

    xw = _xw_matmul(x, W)
    partial = _sc_edge_scatter(xw, srcr, dstr, kidr, zrows)
    s, stats = _sum_and_stats(partial)
    return _bn_relu(s, stats, gamma.reshape(1, C), beta.reshape(1, C))


# speedup vs baseline: 3.0338x; 1.2198x over previous
import functools

import jax
import jax.numpy as jnp
from jax import lax
from jax.experimental import pallas as pl
from jax.experimental.pallas import tpu as pltpu
from jax.experimental.pallas import tpu_sc as plsc

N = 10000
E = 320000
C = 128
K = 27
EPS = 1e-5

NUM_CORES = 2
NUM_SUBCORES = 16
NW = NUM_CORES * NUM_SUBCORES
EPW = E // NW
CHUNK = 80
NCH = EPW // CHUNK
BCH = 25
NBLK_E = NCH // BCH
ZB = 80
NZB = N // ZB

BN_ROWS = 1000
NBLK = N // BN_ROWS


def _xw_matmul(x, W):
    def body(x_ref, w_ref, o_ref):
        o_ref[...] = jnp.dot(x_ref[...], w_ref[0],
                             preferred_element_type=jnp.float32)

    return pl.pallas_call(
        body,
        grid=(NBLK, K),
        in_specs=[
            pl.BlockSpec((BN_ROWS, C), lambda b, k: (b, 0)),
            pl.BlockSpec((1, C, C), lambda b, k: (k, 0, 0)),
        ],
        out_specs=pl.BlockSpec((BN_ROWS, C), lambda b, k: (k * NBLK + b, 0)),
        out_shape=jax.ShapeDtypeStruct((K * N, C), jnp.float32),
    )(x, W)


def _sc_edge_scatter(xw, srcr, dstr, kidr, zrows):
    mesh = plsc.VectorSubcoreMesh(core_axis_name="c", subcore_axis_name="s")

    @functools.partial(
        pl.kernel,
        mesh=mesh,
        out_type=jax.ShapeDtypeStruct((NUM_CORES, N, C), jnp.float32),
        scratch_types=[
            pltpu.VMEM((BCH, CHUNK), jnp.int32),
            pltpu.VMEM((BCH, CHUNK), jnp.int32),
            pltpu.VMEM((BCH, CHUNK), jnp.int32),
            pltpu.VMEM((CHUNK, C), jnp.float32),
            pltpu.VMEM((CHUNK, C), jnp.float32),
            pltpu.VMEM_SHARED((N, C), jnp.float32),
            pltpu.SemaphoreType.DMA,
            pltpu.SemaphoreType.DMA,
        ],
    )
    def k(xw_hbm, src_hbm, dst_hbm, kid_hbm, z_hbm, out_hbm,
          src2, dst2, kid2, rows_a, rows_b, acc, sem_a, sem_b):
        cid = lax.axis_index("c")
        sid = lax.axis_index("s")
        wid = sid * NUM_CORES + cid

        for t in range(NZB // NUM_SUBCORES + 1):
            b = sid + t * NUM_SUBCORES

            @pl.when(b < NZB)
            def _():
                pltpu.sync_copy(z_hbm.at[pl.ds(b * ZB, ZB)],
                                acc.at[pl.ds(b * ZB, ZB)])

        plsc.subcore_barrier()

        def block(bi, carry0):
            pltpu.sync_copy(src_hbm.at[wid, bi], src2)
            pltpu.sync_copy(dst_hbm.at[wid, bi], dst2)
            pltpu.sync_copy(kid_hbm.at[wid, bi], kid2)

            def gidx(i, carry):
                for j in range(CHUNK // 16):
                    sl = pl.ds(j * 16, 16)
                    src2[i, sl] = kid2[i, sl] * N + src2[i, sl]
                return carry
            lax.fori_loop(0, BCH, gidx, 0)

            def gather(ci, rows, sem):
                return pltpu.make_async_copy(xw_hbm.at[src2.at[ci]], rows, sem)

            gather(0, rows_a, sem_a).start()

            def step(t, carry):
                c = 2 * t
                gather(c + 1, rows_b, sem_b).start()
                gather(c, rows_a, sem_a).wait()
                pltpu.sync_copy(rows_a, acc.at[dst2.at[c]], add=True)
                gather(c + 2, rows_a, sem_a).start()
                gather(c + 1, rows_b, sem_b).wait()
                pltpu.sync_copy(rows_b, acc.at[dst2.at[c + 1]], add=True)
                return carry
            lax.fori_loop(0, (BCH - 1) // 2, step, 0)

            gather(BCH - 1, rows_a, sem_a).wait()
            pltpu.sync_copy(rows_a, acc.at[dst2.at[BCH - 1]], add=True)
            return carry0
        lax.fori_loop(0, NBLK_E, block, 0)

        plsc.subcore_barrier()

        for t in range(NZB // NUM_SUBCORES + 1):
            b = sid + t * NUM_SUBCORES

            @pl.when(b < NZB)
            def _():
                pltpu.sync_copy(acc.at[pl.ds(b * ZB, ZB)],
                                out_hbm.at[cid, pl.ds(b * ZB, ZB)])

    return k(xw, srcr, dstr, kidr, zrows)


def _sum_and_stats(partial):
    def body(p_ref, s_ref, st_ref):
        b = pl.program_id(0)
        s = p_ref[0] + p_ref[1]
        s_ref[...] = s

        @pl.when(b == 0)
        def _():
            st_ref[...] = jnp.zeros_like(st_ref)

        st_ref[0, :] += jnp.sum(s, axis=0)
        st_ref[1, :] += jnp.sum(s * s, axis=0)

    return pl.pallas_call(
        body,
        grid=(NBLK,),
        in_specs=[pl.BlockSpec((NUM_CORES, BN_ROWS, C), lambda b: (0, b, 0))],
        out_specs=[
            pl.BlockSpec((BN_ROWS, C), lambda b: (b, 0)),
            pl.BlockSpec((8, C), lambda b: (0, 0)),
        ],
        out_shape=[
            jax.ShapeDtypeStruct((N, C), jnp.float32),
            jax.ShapeDtypeStruct((8, C), jnp.float32),
        ],
    )(partial)


def _bn_relu(s, stats, gamma2, beta2):
    def body(s_ref, st_ref, g_ref, b_ref, o_ref):
        mean = st_ref[0:1, :] * (1.0 / N)
        var = st_ref[1:2, :] * (1.0 / N) - mean * mean
        scale = g_ref[...] * lax.rsqrt(var + EPS)
        o_ref[...] = jnp.maximum((s_ref[...] - mean) * scale + b_ref[...], 0.0)

    return pl.pallas_call(
        body,
        grid=(NBLK,),
        in_specs=[
            pl.BlockSpec((BN_ROWS, C), lambda b: (b, 0)),
            pl.BlockSpec((8, C), lambda b: (0, 0)),
            pl.BlockSpec((1, C), lambda b: (0, 0)),
            pl.BlockSpec((1, C), lambda b: (0, 0)),
        ],
        out_specs=pl.BlockSpec((BN_ROWS, C), lambda b: (b, 0)),
        out_shape=jax.ShapeDtypeStruct((N, C), jnp.float32),
    )(s, stats, gamma2, beta2)


def kernel(x, edge_index, kernel_id, W, gamma, beta):
    srcr = edge_index[0].astype(jnp.int32).reshape(NW, NBLK_E, BCH, CHUNK)
    dstr = edge_index[1].astype(jnp.int32).reshape(NW, NBLK_E, BCH, CHUNK)
    kidr = kernel_id.astype(jnp.int32).reshape(NW, NBLK_E, BCH, CHUNK)
    zrows = jnp.zeros((N, C), jnp.float32)

    xw = _xw_matmul(x, W)
    partial = _sc_edge_scatter(xw, srcr, dstr, kidr, zrows)
    s, stats = _sum_and_stats(partial)
    return _bn_relu(s, stats, gamma.reshape(1, C), beta.reshape(1, C))


# speedup vs baseline: 4.1627x; 1.3721x over previous
import functools

import jax
import jax.numpy as jnp
from jax import lax
from jax.experimental import pallas as pl
from jax.experimental.pallas import tpu as pltpu
from jax.experimental.pallas import tpu_sc as plsc

N = 10000
E = 320000
C = 128
K = 27
EPS = 1e-5

NUM_CORES = 2
NUM_SUBCORES = 16
NW = NUM_CORES * NUM_SUBCORES
EPW = E // NW
CHUNK = 80
NCH = EPW // CHUNK
BCH = 25
NBLK_E = NCH // BCH
ZB = 80
NZB = N // ZB

BN_ROWS = 1000
NBLK = N // BN_ROWS
MM_ROWS = 2000
NBLK_MM = N // MM_ROWS


def _xw_matmul(x, W):
    def body(x_ref, w_ref, o_ref):
        o_ref[...] = jnp.dot(x_ref[...], w_ref[0],
                             preferred_element_type=jnp.float32
)

    return pl.pallas_call(
        body,
        grid=(NBLK_MM, K),
        in_specs=[
            pl.BlockSpec((MM_ROWS, C), lambda b, k: (b, 0)),
            pl.BlockSpec((1, C, C), lambda b, k: (k, 0, 0)),
        ],
        out_specs=pl.BlockSpec((MM_ROWS, C),
                               lambda b, k: (k * NBLK_MM + b, 0)),
        out_shape=jax.ShapeDtypeStruct((K * N, C), jnp.float32),
    )(x, W)


def _sc_edge_scatter(xw, pr, dstr):
    mesh = plsc.VectorSubcoreMesh(core_axis_name="c", subcore_axis_name="s")

    @functools.partial(
        pl.kernel,
        mesh=mesh,
        out_type=jax.ShapeDtypeStruct((NUM_CORES, N, C), jnp.float32),
        scratch_types=[
            pltpu.VMEM((BCH, CHUNK), jnp.int32),
            pltpu.VMEM((BCH, CHUNK), jnp.int32),
            pltpu.VMEM((BCH, CHUNK), jnp.int32),
            pltpu.VMEM((BCH, CHUNK), jnp.int32),
            pltpu.VMEM((CHUNK, C), jnp.float32),
            pltpu.VMEM((CHUNK, C), jnp.float32),
            pltpu.VMEM_SHARED((N, C), jnp.float32),
            pltpu.SemaphoreType.DMA,
            pltpu.SemaphoreType.DMA,
            pltpu.SemaphoreType.DMA,
            pltpu.SemaphoreType.DMA,
        ],
    )
    def k(p_hbm, dst_hbm, xw_hbm, out_hbm,
          p2a, p2b, d2a, d2b, rows_a, rows_b, acc,
          sem_a, sem_b, sem_ma, sem_mb):
        cid = lax.axis_index("c")
        sid = lax.axis_index("s")
        wid = sid * NUM_CORES + cid

        pltpu.make_async_copy(p_hbm.at[wid, 0], p2a, sem_ma).start()
        pltpu.make_async_copy(dst_hbm.at[wid, 0], d2a, sem_ma).start()

        def zrow(i, carry):
            for j in range(C // 16):
                rows_a[i, pl.ds(j * 16, 16)] = jnp.zeros((16,), jnp.float32)
            return carry
        lax.fori_loop(0, CHUNK, zrow, 0)
        for t in range(NZB // NUM_SUBCORES + 1):
            b = sid + t * NUM_SUBCORES

            @pl.when(b < NZB)
            def _():
                pltpu.sync_copy(rows_a, acc.at[pl.ds(b * ZB, ZB)])

        plsc.subcore_barrier()

        for bi in range(NBLK_E):
            if bi % 2 == 0:
                pb, db, sm = p2a, d2a, sem_ma
                pn, dn, smn = p2b, d2b, sem_mb
            else:
                pb, db, sm = p2b, d2b, sem_mb
                pn, dn, smn = p2a, d2a, sem_ma

            pltpu.make_async_copy(p_hbm.at[wid, bi], pb, sm).wait()
            pltpu.make_async_copy(dst_hbm.at[wid, bi], db, sm).wait()
            if bi + 1 < NBLK_E:
                pltpu.make_async_copy(p_hbm.at[wid, bi + 1], pn, smn).start()
                pltpu.make_async_copy(dst_hbm.at[wid, bi + 1], dn, smn).start()

            def gidx(i, carry):
                for j in range(CHUNK // 16):
                    sl = pl.ds(j * 16, 16)
                    v = pb[i, sl]
                    pb[i, sl] = (v >> 14) * N + (v & 16383)
                return carry
            lax.fori_loop(0, BCH, gidx, 0)

            def gather(ci, rows, sem, pb=pb):
                return pltpu.make_async_copy(xw_hbm.at[pb.at[ci]], rows, sem)

            gather(0, rows_a, sem_a).start()

            def step(t, carry, gather=gather, db=db):
                c = 2 * t
                gather(c + 1, rows_b, sem_b).start()
                gather(c, rows_a, sem_a).wait()
                pltpu.sync_copy(rows_a, acc.at[db.at[c]], add=True)
                gather(c + 2, rows_a, sem_a).start()
                gather(c + 1, rows_b, sem_b).wait()
                pltpu.sync_copy(rows_b, acc.at[db.at[c + 1]], add=True)
                return carry
            lax.fori_loop(0, (BCH - 1) // 2, step, 0)

            gather(BCH - 1, rows_a, sem_a).wait()
            pltpu.sync_copy(rows_a, acc.at[db.at[BCH - 1]], add=True)

        plsc.subcore_barrier()

        for t in range(NZB // NUM_SUBCORES + 1):
            b = sid + t * NUM_SUBCORES

            @pl.when(b < NZB)
            def _():
                pltpu.sync_copy(acc.at[pl.ds(b * ZB, ZB)],
                                out_hbm.at[cid, pl.ds(b * ZB, ZB)])

    return k(pr, dstr, xw)


def _sum_and_stats(partial):
    def body(p_ref, s_ref, st_ref):
        b = pl.program_id(0)
        s = (p_ref[0].astype(jnp.float32) + p_ref[1].astype(jnp.float32))
        s_ref[...] = s

        @pl.when(b == 0)
        def _():
            st_ref[...] = jnp.zeros_like(st_ref)

        st_ref[0, :] += jnp.sum(s, axis=0)
        st_ref[1, :] += jnp.sum(s * s, axis=0)

    return pl.pallas_call(
        body,
        grid=(NBLK_MM,),
        in_specs=[pl.BlockSpec((NUM_CORES, MM_ROWS, C), lambda b: (0, b, 0))],
        out_specs=[
            pl.BlockSpec((MM_ROWS, C), lambda b: (b, 0)),
            pl.BlockSpec((8, C), lambda b: (0, 0)),
        ],
        out_shape=[
            jax.ShapeDtypeStruct((N, C), jnp.float32),
            jax.ShapeDtypeStruct((8, C), jnp.float32),
        ],
    )(partial)


def _bn_relu(s, stats, gamma2, beta2):
    def body(s_ref, st_ref, g_ref, b_ref, o_ref):
        mean = st_ref[0:1, :] * (1.0 / N)
        var = st_ref[1:2, :] * (1.0 / N) - mean * mean
        scale = g_ref[...] * lax.rsqrt(var + EPS)
        o_ref[...] = jnp.maximum((s_ref[...] - mean) * scale + b_ref[...], 0.0)

    return pl.pallas_call(
        body,
        grid=(NBLK,),
        in_specs=[
            pl.BlockSpec((BN_ROWS, C), lambda b: (b, 0)),
            pl.BlockSpec((8, C), lambda b: (0, 0)),
            pl.BlockSpec((1, C), lambda b: (0, 0)),
            pl.BlockSpec((1, C), lambda b: (0, 0)),
        ],
        out_specs=pl.BlockSpec((BN_ROWS, C), lambda b: (b, 0)),
        out_shape=jax.ShapeDtypeStruct((N, C), jnp.float32),
    )(s, stats, gamma2, beta2)


def kernel(x, edge_index, kernel_id, W, gamma, beta):
    src = edge_index[0].astype(jnp.int32)
    kid = kernel_id.astype(jnp.int32)
    pr = ((kid << 14) | src).reshape(NW, NBLK_E, BCH, CHUNK)
    dstr = edge_index[1].astype(jnp.int32).reshape(NW, NBLK_E, BCH, CHUNK)

    xw = _xw_matmul(x, W)
    partial = _sc_edge_scatter(xw, pr, dstr)
    s, stats = _sum_and_stats(partial)
    return _bn_relu(s, stats, gamma.reshape(1, C), beta.reshape(1, C))


# speedup vs baseline: 5.0596x; 1.2154x over previous
import functools

import jax
import jax.numpy as jnp
from jax import lax
from jax.experimental import pallas as pl
from jax.experimental.pallas import tpu as pltpu
from jax.experimental.pallas import tpu_sc as plsc

N = 10000
E = 320000
C = 128
K = 27
EPS = 1e-5

NUM_CORES = 2
NUM_SUBCORES = 16
NW = NUM_CORES * NUM_SUBCORES
EPW = E // NW
CHUNK = 80
NCH = EPW // CHUNK
BCH = 25
NBLK_E = NCH // BCH
ZB = 80
NZB = N // ZB

BN_ROWS = 1000
NBLK = N // BN_ROWS
MM_ROWS = 5000
NBLK_MM = N // MM_ROWS
ST_ROWS = 2000
NBLK_ST = N // ST_ROWS


def _xw_matmul(x, W):
    def body(x_ref, w_ref, o_ref):
        o_ref[...] = jnp.dot(x_ref[...], w_ref[0],
                             preferred_element_type=jnp.float32
)

    return pl.pallas_call(
        body,
        grid=(NBLK_MM, K),
        in_specs=[
            pl.BlockSpec((MM_ROWS, C), lambda b, k: (b, 0)),
            pl.BlockSpec((1, C, C), lambda b, k: (k, 0, 0)),
        ],
        out_specs=pl.BlockSpec((MM_ROWS, C),
                               lambda b, k: (k * NBLK_MM + b, 0)),
        out_shape=jax.ShapeDtypeStruct((K * N, C), jnp.float32),
    )(x, W)


def _sc_edge_scatter(xw, er, kidr):
    mesh = plsc.VectorSubcoreMesh(core_axis_name="c", subcore_axis_name="s")

    @functools.partial(
        pl.kernel,
        mesh=mesh,
        out_type=jax.ShapeDtypeStruct((NUM_CORES, N, C), jnp.float32),
        scratch_types=[
            pltpu.VMEM((BCH, CHUNK), jnp.int32),
            pltpu.VMEM((BCH, CHUNK), jnp.int32),
            pltpu.VMEM((BCH, CHUNK), jnp.int32),
            pltpu.VMEM((BCH, CHUNK), jnp.int32),
            pltpu.VMEM((BCH, CHUNK), jnp.int32),
            pltpu.VMEM((CHUNK, C), jnp.float32),
            pltpu.VMEM((CHUNK, C), jnp.float32),
            pltpu.VMEM_SHARED((N, C), jnp.float32),
            pltpu.SemaphoreType.DMA,
            pltpu.SemaphoreType.DMA,
            pltpu.SemaphoreType.DMA,
            pltpu.SemaphoreType.DMA,
            pltpu.SemaphoreType.DMA,
        ],
    )
    def k(er_hbm, kid_hbm, xw_hbm, out_hbm,
          p2a, p2b, d2a, d2b, kid1, rows_a, rows_b, acc,
          sem_a, sem_b, sem_ma, sem_mb, sem_k):
        cid = lax.axis_index("c")
        sid = lax.axis_index("s")
        wid = sid * NUM_CORES + cid

        pltpu.make_async_copy(er_hbm.at[0, wid, 0], p2a, sem_ma).start()
        pltpu.make_async_copy(er_hbm.at[1, wid, 0], d2a, sem_ma).start()
        pltpu.make_async_copy(kid_hbm.at[wid, 0], kid1, sem_k).start()

        def zrow(i, carry):
            for j in range(C // 16):
                rows_a[i, pl.ds(j * 16, 16)] = jnp.zeros((16,), jnp.float32)
            return carry
        lax.fori_loop(0, CHUNK, zrow, 0)
        for t in range(NZB // NUM_SUBCORES + 1):
            b = sid + t * NUM_SUBCORES

            @pl.when(b < NZB)
            def _():
                pltpu.sync_copy(rows_a, acc.at[pl.ds(b * ZB, ZB)])

        plsc.subcore_barrier()

        for bi in range(NBLK_E):
            if bi % 2 == 0:
                pb, db, sm = p2a, d2a, sem_ma
                pn, dn, smn = p2b, d2b, sem_mb
            else:
                pb, db, sm = p2b, d2b, sem_mb
                pn, dn, smn = p2a, d2a, sem_ma

            pltpu.make_async_copy(er_hbm.at[0, wid, bi], pb, sm).wait()
            pltpu.make_async_copy(er_hbm.at[1, wid, bi], db, sm).wait()
            pltpu.make_async_copy(kid_hbm.at[wid, bi], kid1, sem_k).wait()
            if bi + 1 < NBLK_E:
                pltpu.make_async_copy(er_hbm.at[0, wid, bi + 1], pn,
                                      smn).start()
                pltpu.make_async_copy(er_hbm.at[1, wid, bi + 1], dn,
                                      smn).start()

            def gidx(i, carry):
                for j in range(CHUNK // 16):
                    sl = pl.ds(j * 16, 16)
                    pb[i, sl] = kid1[i, sl] * N + pb[i, sl]
                return carry
            lax.fori_loop(0, BCH, gidx, 0)

            if bi + 1 < NBLK_E:
                pltpu.make_async_copy(kid_hbm.at[wid, bi + 1], kid1,
                                      sem_k).start()

            def gather(ci, rows, sem, pb=pb):
                return pltpu.make_async_copy(xw_hbm.at[pb.at[ci]], rows, sem)

            gather(0, rows_a, sem_a).start()

            def step(t, carry, gather=gather, db=db):
                c = 2 * t
                gather(c + 1, rows_b, sem_b).start()
                gather(c, rows_a, sem_a).wait()
                pltpu.sync_copy(rows_a, acc.at[db.at[c]], add=True)
                gather(c + 2, rows_a, sem_a).start()
                gather(c + 1, rows_b, sem_b).wait()
                pltpu.sync_copy(rows_b, acc.at[db.at[c + 1]], add=True)
                return carry
            lax.fori_loop(0, (BCH - 1) // 2, step, 0)

            gather(BCH - 1, rows_a, sem_a).wait()
            pltpu.sync_copy(rows_a, acc.at[db.at[BCH - 1]], add=True)

        plsc.subcore_barrier()

        for t in range(NZB // NUM_SUBCORES + 1):
            b = sid + t * NUM_SUBCORES

            @pl.when(b < NZB)
            def _():
                pltpu.sync_copy(acc.at[pl.ds(b * ZB, ZB)],
                                out_hbm.at[cid, pl.ds(b * ZB, ZB)])

    return k(er, kidr, xw)


def _sum_and_stats(partial):
    def body(p_ref, s_ref, st_ref):
        b = pl.program_id(0)
        s = (p_ref[0].astype(jnp.float32) + p_ref[1].astype(jnp.float32))
        s_ref[...] = s

        @pl.when(b == 0)
        def _():
            st_ref[...] = jnp.zeros_like(st_ref)

        st_ref[0, :] += jnp.sum(s, axis=0)
        st_ref[1, :] += jnp.sum(s * s, axis=0)

    return pl.pallas_call(
        body,
        grid=(NBLK_ST,),
        in_specs=[pl.BlockSpec((NUM_CORES, ST_ROWS, C), lambda b: (0, b, 0))],
        out_specs=[
            pl.BlockSpec((ST_ROWS, C), lambda b: (b, 0)),
            pl.BlockSpec((8, C), lambda b: (0, 0)),
        ],
        out_shape=[
            jax.ShapeDtypeStruct((N, C), jnp.float32),
            jax.ShapeDtypeStruct((8, C), jnp.float32),
        ],
    )(partial)


def _bn_relu(s, stats, gamma2, beta2):
    def body(s_ref, st_ref, g_ref, b_ref, o_ref):
        mean = st_ref[0:1, :] * (1.0 / N)
        var = st_ref[1:2, :] * (1.0 / N) - mean * mean
        scale = g_ref[...] * lax.rsqrt(var + EPS)
        o_ref[...] = jnp.maximum((s_ref[...] - mean) * scale + b_ref[...], 0.0)

    return pl.pallas_call(
        body,
        grid=(NBLK,),
        in_specs=[
            pl.BlockSpec((BN_ROWS, C), lambda b: (b, 0)),
            pl.BlockSpec((8, C), lambda b: (0, 0)),
            pl.BlockSpec((1, C), lambda b: (0, 0)),
            pl.BlockSpec((1, C), lambda b: (0, 0)),
        ],
        out_specs=pl.BlockSpec((BN_ROWS, C), lambda b: (b, 0)),
        out_shape=jax.ShapeDtypeStruct((N, C), jnp.float32),
    )(s, stats, gamma2, beta2)


def kernel(x, edge_index, kernel_id, W, gamma, beta):
    er = edge_index.astype(jnp.int32).reshape(2, NW, NBLK_E, BCH, CHUNK)
    kidr = kernel_id.astype(jnp.int32).reshape(NW, NBLK_E, BCH, CHUNK)

    xw = _xw_matmul(x, W)
    partial = _sc_edge_scatter(xw, er, kidr)
    s, stats = _sum_and_stats(partial)
    return _bn_relu(s, stats, gamma.reshape(1, C), beta.reshape(1, C))
